# baseline (speedup 1.0000x reference)
import functools

import jax
import jax.numpy as jnp
from jax.experimental import pallas as pl

TOPK = 8
NEXP = 64
BLK = 512


def _gate_kernel(x_ref, w_ref, ow_ref, oi_ref):
    x = x_ref[...]
    w = w_ref[...]
    logits = jax.lax.dot_general(
        x, w,
        dimension_numbers=(((1,), (1,)), ((), ())),
        preferred_element_type=jnp.float32,
    )
    b = logits.shape[0]
    lane = jax.lax.broadcasted_iota(jnp.int32, (b, NEXP), 1)
    vals = logits
    top_vals = []
    top_idxs = []
    for _ in range(TOPK):
        m = jnp.max(vals, axis=-1, keepdims=True)
        idx = jnp.argmax(vals, axis=-1, keepdims=True).astype(jnp.int32)
        top_vals.append(m)
        top_idxs.append(idx)
        vals = jnp.where(lane == idx, -jnp.inf, vals)
    tv = jnp.concatenate(top_vals, axis=1)
    ti = jnp.concatenate(top_idxs, axis=1)
    e = jnp.exp(tv - tv[:, :1])
    ow_ref[...] = e / jnp.sum(e, axis=-1, keepdims=True)
    oi_ref[...] = ti


@functools.partial(jax.jit, static_argnames=())
def kernel(x, W):
    n, d = x.shape
    grid = (n // BLK,)
    ow, oi = pl.pallas_call(
        _gate_kernel,
        grid=grid,
        in_specs=[
            pl.BlockSpec((BLK, d), lambda i: (i, 0)),
            pl.BlockSpec((NEXP, d), lambda i: (0, 0)),
        ],
        out_specs=[
            pl.BlockSpec((BLK, TOPK), lambda i: (i, 0)),
            pl.BlockSpec((BLK, TOPK), lambda i: (i, 0)),
        ],
        out_shape=[
            jax.ShapeDtypeStruct((n, TOPK), jnp.float32),
            jax.ShapeDtypeStruct((n, TOPK), jnp.int32),
        ],
    )(x, W)
    return ow.astype(x.dtype), oi


# speedup vs baseline: 1.0254x; 1.0254x over previous
import functools

import jax
import jax.numpy as jnp
from jax.experimental import pallas as pl

TOPK = 8
NEXP = 64
BLK = 1024


def _gate_kernel(x_ref, w_ref, ow_ref, oi_ref):
    x = x_ref[...]
    w = w_ref[...]
    logits = jax.lax.dot_general(
        x, w,
        dimension_numbers=(((1,), (1,)), ((), ())),
        preferred_element_type=jnp.float32,
    )
    b = logits.shape[0]
    lane = jax.lax.broadcasted_iota(jnp.int32, (b, NEXP), 1)
    vals = logits
    top_vals = []
    top_idxs = []
    for _ in range(TOPK):
        m = jnp.max(vals, axis=-1, keepdims=True)
        idx = jnp.argmax(vals, axis=-1, keepdims=True).astype(jnp.int32)
        top_vals.append(m)
        top_idxs.append(idx)
        vals = jnp.where(lane == idx, -jnp.inf, vals)
    tv = jnp.concatenate(top_vals, axis=1)
    ti = jnp.concatenate(top_idxs, axis=1)
    e = jnp.exp(tv - tv[:, :1])
    ow_ref[...] = e / jnp.sum(e, axis=-1, keepdims=True)
    oi_ref[...] = ti


@functools.partial(jax.jit, static_argnames=())
def kernel(x, W):
    n, d = x.shape
    grid = (n // BLK,)
    ow, oi = pl.pallas_call(
        _gate_kernel,
        grid=grid,
        in_specs=[
            pl.BlockSpec((BLK, d), lambda i: (i, 0)),
            pl.BlockSpec((NEXP, d), lambda i: (0, 0)),
        ],
        out_specs=[
            pl.BlockSpec((BLK, TOPK), lambda i: (i, 0)),
            pl.BlockSpec((BLK, TOPK), lambda i: (i, 0)),
        ],
        out_shape=[
            jax.ShapeDtypeStruct((n, TOPK), jnp.float32),
            jax.ShapeDtypeStruct((n, TOPK), jnp.int32),
        ],
    )(x, W)
    return ow.astype(x.dtype), oi


# speedup vs baseline: 1.1190x; 1.0913x over previous
import functools

import jax
import jax.numpy as jnp
from jax.experimental import pallas as pl

TOPK = 8
NEXP = 64
BLK = 1024


def _gate_kernel(x_ref, w_ref, ow_ref, oi_ref):
    x = x_ref[...]
    w = w_ref[...]
    logits = jax.lax.dot_general(
        x, w,
        dimension_numbers=(((1,), (1,)), ((), ())),
        preferred_element_type=jnp.float32,
    )
    b = logits.shape[0]
    lane = jax.lax.broadcasted_iota(jnp.int32, (b, NEXP), 1)
    bits = jax.lax.bitcast_convert_type(logits, jnp.int32)
    code = jnp.where(bits < 0, lane, lane ^ 63)
    key = jax.lax.bitcast_convert_type(
        (bits & jnp.int32(~63)) | code, jnp.float32)
    top_keys = []
    for _ in range(TOPK):
        m = jnp.max(key, axis=-1, keepdims=True)
        top_keys.append(m)
        key = jnp.where(key == m, -jnp.inf, key)
    tk = jnp.concatenate(top_keys, axis=1)
    kb = jax.lax.bitcast_convert_type(tk, jnp.int32)
    kc = kb & jnp.int32(63)
    ti = jnp.where(kb < 0, kc, kc ^ 63)
    tv = jax.lax.bitcast_convert_type(kb & jnp.int32(~63), jnp.float32)
    e = jnp.exp(tv - tv[:, :1])
    ow_ref[...] = e / jnp.sum(e, axis=-1, keepdims=True)
    oi_ref[...] = ti


@functools.partial(jax.jit, static_argnames=())
def kernel(x, W):
    n, d = x.shape
    grid = (n // BLK,)
    ow, oi = pl.pallas_call(
        _gate_kernel,
        grid=grid,
        in_specs=[
            pl.BlockSpec((BLK, d), lambda i: (i, 0)),
            pl.BlockSpec((NEXP, d), lambda i: (0, 0)),
        ],
        out_specs=[
            pl.BlockSpec((BLK, TOPK), lambda i: (i, 0)),
            pl.BlockSpec((BLK, TOPK), lambda i: (i, 0)),
        ],
        out_shape=[
            jax.ShapeDtypeStruct((n, TOPK), jnp.float32),
            jax.ShapeDtypeStruct((n, TOPK), jnp.int32),
        ],
    )(x, W)
    return ow.astype(x.dtype), oi


# speedup vs baseline: 1.2222x; 1.0923x over previous
import functools

import jax
import jax.numpy as jnp
from jax.experimental import pallas as pl

TOPK = 8
NEXP = 64
BLK = 1024


SUB = 4


def _topk_part(logits):
    b = logits.shape[0]
    lane = jax.lax.broadcasted_iota(jnp.int32, (b, NEXP), 1)
    bits = jax.lax.bitcast_convert_type(logits, jnp.int32)
    code = jnp.where(bits < 0, lane, lane ^ 63)
    key = jax.lax.bitcast_convert_type(
        (bits & jnp.int32(~63)) | code, jnp.float32)
    top_keys = []
    for _ in range(TOPK):
        m = jnp.max(key, axis=-1, keepdims=True)
        top_keys.append(m)
        key = jnp.where(key == m, -jnp.inf, key)
    tk = jnp.concatenate(top_keys, axis=1)
    kb = jax.lax.bitcast_convert_type(tk, jnp.int32)
    kc = kb & jnp.int32(63)
    ti = jnp.where(kb < 0, kc, kc ^ 63)
    tv = jax.lax.bitcast_convert_type(kb & jnp.int32(~63), jnp.float32)
    e = jnp.exp(tv - tv[:, :1])
    return e / jnp.sum(e, axis=-1, keepdims=True), ti


def _gate_kernel(x_ref, w_ref, ow_ref, oi_ref):
    w = w_ref[...]
    c = BLK // SUB
    for i in range(SUB):
        sl = pl.ds(i * c, c)
        logits = jax.lax.dot_general(
            x_ref[sl, :], w,
            dimension_numbers=(((1,), (1,)), ((), ())),
            preferred_element_type=jnp.float32,
        )
        ow, oi = _topk_part(logits)
        ow_ref[sl, :] = ow
        oi_ref[sl, :] = oi


@functools.partial(jax.jit, static_argnames=())
def kernel(x, W):
    n, d = x.shape
    grid = (n // BLK,)
    ow, oi = pl.pallas_call(
        _gate_kernel,
        grid=grid,
        in_specs=[
            pl.BlockSpec((BLK, d), lambda i: (i, 0)),
            pl.BlockSpec((NEXP, d), lambda i: (0, 0)),
        ],
        out_specs=[
            pl.BlockSpec((BLK, TOPK), lambda i: (i, 0)),
            pl.BlockSpec((BLK, TOPK), lambda i: (i, 0)),
        ],
        out_shape=[
            jax.ShapeDtypeStruct((n, TOPK), jnp.float32),
            jax.ShapeDtypeStruct((n, TOPK), jnp.int32),
        ],
    )(x, W)
    return ow.astype(x.dtype), oi
